# Initial kernel scaffold; baseline (speedup 1.0000x reference)
import jax
import jax.numpy as jnp
from jax.experimental import pallas as pl


def kernel(x, edge_index, W1l, b1, W1r, W2l, b2, W2r, Wout, bout):
    raise NotImplementedError("write your pallas kernel here")


# speedup vs baseline: 5.4716x; 5.4716x over previous
import functools

import jax
import jax.numpy as jnp
from jax import lax
from jax.experimental import pallas as pl
from jax.experimental.pallas import tpu as pltpu
from jax.experimental.pallas import tpu_sc as plsc

N_NODES = 10000
N_EDGES = 320000
D = 128

NC = 2
NS = 16
NW = NC * NS
N_PAD = 10240
RPT = N_PAD // NS
EPW = N_EDGES // NW
WIN = 80
NWIN = EPW // WIN


def _sc_aggregate_body(with_cnt, src_ref, dst_ref, h_ref, zrow_ref, zcnt_ref,
                       agg_out, cnt_out, idx_s, idx_d, rows, ones, acc,
                       cacc, sem):
  c = lax.axis_index("c")
  s = lax.axis_index("s")
  wid = c * NS + s

  pltpu.sync_copy(zrow_ref.at[pl.ds(s * RPT, RPT)],
                  acc.at[pl.ds(s * RPT, RPT)])
  if with_cnt:
    pltpu.sync_copy(zcnt_ref.at[pl.ds(s * RPT, RPT)],
                    cacc.at[pl.ds(s * RPT, RPT)])
    def fill(i, _):
      ones[pl.ds(i * 16, 16)] = jnp.full((16,), 1.0, jnp.float32)
      return 0
    lax.fori_loop(0, WIN // 16, fill, 0)
  plsc.subcore_barrier()

  e0 = wid * EPW

  def step(w, _):
    base = e0 + w * WIN
    pltpu.sync_copy(src_ref.at[pl.ds(base, WIN)], idx_s)
    pltpu.sync_copy(dst_ref.at[pl.ds(base, WIN)], idx_d)
    pltpu.async_copy(h_ref.at[idx_s], rows, sem).wait()
    pltpu.sync_copy(rows, acc.at[idx_d], add=True)
    if with_cnt:
      pltpu.sync_copy(ones, cacc.at[idx_d], add=True)
    return 0

  lax.fori_loop(0, NWIN, step, 0)
  plsc.subcore_barrier()

  pltpu.sync_copy(acc.at[pl.ds(s * RPT, RPT)],
                  agg_out.at[c, pl.ds(s * RPT, RPT)])
  if with_cnt:
    pltpu.sync_copy(cacc.at[pl.ds(s * RPT, RPT)],
                    cnt_out.at[c, pl.ds(s * RPT, RPT)])


def _make_sc_aggregate(with_cnt):
  mesh = plsc.VectorSubcoreMesh(core_axis_name="c", subcore_axis_name="s")
  out_type = (
      jax.ShapeDtypeStruct((NC, N_PAD, D), jnp.float32),
      jax.ShapeDtypeStruct((NC, N_PAD), jnp.float32),
  )
  scratch = [
      pltpu.VMEM((WIN,), jnp.int32),
      pltpu.VMEM((WIN,), jnp.int32),
      pltpu.VMEM((WIN, D), jnp.float32),
      pltpu.VMEM((WIN,), jnp.float32),
      pltpu.VMEM_SHARED((N_PAD, D), jnp.float32),
      pltpu.VMEM_SHARED((N_PAD,), jnp.float32),
      pltpu.SemaphoreType.DMA,
  ]
  return pl.kernel(
      functools.partial(_sc_aggregate_body, with_cnt),
      out_type=out_type,
      mesh=mesh,
      scratch_types=scratch,
      name="sage_sc_aggregate",
  )


_sc_aggregate_cnt = _make_sc_aggregate(True)
_sc_aggregate_nocnt = _make_sc_aggregate(False)

BLK = 1280


def _tc_layer1_body(aggp_ref, cntp_ref, x_ref, w1l_ref, b1_ref, w1r_ref,
                    out_ref):
  agg = aggp_ref[0] + aggp_ref[1]
  cnt = cntp_ref[0] + cntp_ref[1]
  inv = 1.0 / jnp.maximum(cnt, 1.0)
  mean = agg * inv
  h = lax.dot_general(mean, w1l_ref[...], (((1,), (1,)), ((), ())),
                      preferred_element_type=jnp.float32)
  h += lax.dot_general(x_ref[...], w1r_ref[...], (((1,), (1,)), ((), ())),
                       preferred_element_type=jnp.float32)
  h += b1_ref[...][None, :]
  out_ref[...] = jnp.maximum(h, 0.0)


def _tc_layer2_body(aggp_ref, cntp_ref, h_ref, w2l_ref, b2_ref, w2r_ref,
                    wout_ref, bout_ref, out_ref):
  agg = aggp_ref[0] + aggp_ref[1]
  cnt = cntp_ref[0] + cntp_ref[1]
  inv = 1.0 / jnp.maximum(cnt, 1.0)
  mean = agg * inv
  h = lax.dot_general(mean, w2l_ref[...], (((1,), (1,)), ((), ())),
                      preferred_element_type=jnp.float32)
  h += lax.dot_general(h_ref[...], w2r_ref[...], (((1,), (1,)), ((), ())),
                       preferred_element_type=jnp.float32)
  h += b2_ref[...][None, :]
  h = jnp.maximum(h, 0.0)
  out = lax.dot_general(h, wout_ref[...], (((1,), (1,)), ((), ())),
                        preferred_element_type=jnp.float32)
  out += bout_ref[...][None, :]
  out_ref[...] = out


def _row_block(i):
  return (0, i, 0)


_agg_spec = pl.BlockSpec((NC, BLK, D), lambda i: (0, i, 0))
_cnt_spec = pl.BlockSpec((NC, BLK, 1), lambda i: (0, i, 0))
_h_spec = pl.BlockSpec((BLK, D), lambda i: (i, 0))
_w_spec = pl.BlockSpec((D, D), lambda i: (0, 0))
_b_spec = pl.BlockSpec((D,), lambda i: (0,))

_tc_layer1 = pl.pallas_call(
    _tc_layer1_body,
    grid=(N_PAD // BLK,),
    in_specs=[_agg_spec, _cnt_spec, _h_spec, _w_spec, _b_spec, _w_spec],
    out_specs=_h_spec,
    out_shape=jax.ShapeDtypeStruct((N_PAD, D), jnp.float32),
)

_tc_layer2 = pl.pallas_call(
    _tc_layer2_body,
    grid=(N_PAD // BLK,),
    in_specs=[_agg_spec, _cnt_spec, _h_spec, _w_spec, _b_spec, _w_spec,
              _w_spec, _b_spec],
    out_specs=_h_spec,
    out_shape=jax.ShapeDtypeStruct((N_PAD, D), jnp.float32),
)


@jax.jit
def kernel(x, edge_index, W1l, b1, W1r, W2l, b2, W2r, Wout, bout):
  edges = edge_index.astype(jnp.int32)
  src = edges[0]
  dst = edges[1]
  xp = jnp.zeros((N_PAD, D), jnp.float32).at[:N_NODES].set(x)
  zrow = jnp.zeros((N_PAD, D), jnp.float32)
  zcnt = jnp.zeros((N_PAD,), jnp.float32)

  aggp1, cntp = _sc_aggregate_cnt(src, dst, xp, zrow, zcnt)
  cntp3 = cntp.reshape(NC, N_PAD, 1)
  h1 = _tc_layer1(aggp1, cntp3, xp, W1l, b1, W1r)

  aggp2, _ = _sc_aggregate_nocnt(src, dst, h1, zrow, zcnt)
  out = _tc_layer2(aggp2, cntp3, h1, W2l, b2, W2r, Wout, bout)
  return out[:N_NODES]


# speedup vs baseline: 7.5930x; 1.3877x over previous
import functools

import jax
import jax.numpy as jnp
from jax import lax
from jax.experimental import pallas as pl
from jax.experimental.pallas import tpu as pltpu
from jax.experimental.pallas import tpu_sc as plsc

N_NODES = 10000
N_EDGES = 320000
D = 128

NC = 2
NS = 16
NW = NC * NS
N_PAD = 10240
RPT = N_PAD // NS
EPW = N_EDGES // NW
WIN = 80
NWIN = 126
EPW_P = NWIN * WIN
NBUF = 2


def _sc_aggregate_body(with_cnt, src_ref, dst_ref, h_ref, zrow_ref, zcnt_ref,
                       agg_out, cnt_out, src_all, dst_all, rows0, rows1,
                       ones, acc, cacc, sem0, sem1):
  c = lax.axis_index("c")
  s = lax.axis_index("s")
  wid = c * NS + s

  pltpu.sync_copy(src_ref.at[wid], src_all)
  pltpu.sync_copy(dst_ref.at[wid], dst_all)
  pltpu.sync_copy(zrow_ref.at[pl.ds(s * RPT, RPT)],
                  acc.at[pl.ds(s * RPT, RPT)])
  if with_cnt:
    pltpu.sync_copy(zcnt_ref.at[pl.ds(s * RPT, RPT)],
                    cacc.at[pl.ds(s * RPT, RPT)])
    def fill(i, _):
      ones[pl.ds(i * 16, 16)] = jnp.full((16,), 1.0, jnp.float32)
      return 0
    lax.fori_loop(0, WIN // 16, fill, 0)
  plsc.subcore_barrier()

  bufs = (rows0, rows1)
  semt = (sem0, sem1)

  def sidx(w):
    return src_all.at[pl.ds(w * WIN, WIN)]

  for b in range(NBUF):
    pltpu.async_copy(h_ref.at[sidx(b)], bufs[b], semt[b])

  @pl.loop(0, NWIN - NBUF, step=NBUF)
  def _(g):
    for b in range(NBUF):
      w = g + b
      pltpu.make_async_copy(h_ref.at[sidx(w)], bufs[b], semt[b]).wait()
      pltpu.sync_copy(bufs[b], acc.at[dst_all.at[w]], add=True)
      if with_cnt:
        pltpu.sync_copy(ones, cacc.at[dst_all.at[w]], add=True)
      pltpu.async_copy(h_ref.at[sidx(w + NBUF)], bufs[b], semt[b])

  for b in range(NBUF):
    w = NWIN - NBUF + b
    pltpu.make_async_copy(h_ref.at[sidx(w)], bufs[b], semt[b]).wait()
    pltpu.sync_copy(bufs[b], acc.at[dst_all.at[w]], add=True)
    if with_cnt:
      pltpu.sync_copy(ones, cacc.at[dst_all.at[w]], add=True)

  plsc.subcore_barrier()

  pltpu.sync_copy(acc.at[pl.ds(s * RPT, RPT)],
                  agg_out.at[c, pl.ds(s * RPT, RPT)])
  if with_cnt:
    pltpu.sync_copy(cacc.at[pl.ds(s * RPT, RPT)],
                    cnt_out.at[c, pl.ds(s * RPT, RPT)])


def _make_sc_aggregate(with_cnt):
  mesh = plsc.VectorSubcoreMesh(core_axis_name="c", subcore_axis_name="s")
  out_type = (
      jax.ShapeDtypeStruct((NC, N_PAD, D), jnp.float32),
      jax.ShapeDtypeStruct((NC, N_PAD), jnp.float32),
  )
  scratch = [
      pltpu.VMEM((EPW_P,), jnp.int32),
      pltpu.VMEM((NWIN, WIN), jnp.int32),
      pltpu.VMEM((WIN, D), jnp.float32),
      pltpu.VMEM((WIN, D), jnp.float32),
      pltpu.VMEM((WIN,), jnp.float32),
      pltpu.VMEM_SHARED((N_PAD, D), jnp.float32),
      pltpu.VMEM_SHARED((N_PAD,), jnp.float32),
      pltpu.SemaphoreType.DMA,
      pltpu.SemaphoreType.DMA,
  ]
  return pl.kernel(
      functools.partial(_sc_aggregate_body, with_cnt),
      out_type=out_type,
      mesh=mesh,
      scratch_types=scratch,
      name="sage_sc_aggregate",
  )


_sc_aggregate_cnt = _make_sc_aggregate(True)
_sc_aggregate_nocnt = _make_sc_aggregate(False)

BLK = 1280


def _tc_layer1_body(aggp_ref, cntp_ref, x_ref, w1l_ref, b1_ref, w1r_ref,
                    out_ref):
  agg = aggp_ref[0] + aggp_ref[1]
  cnt = cntp_ref[0] + cntp_ref[1]
  inv = 1.0 / jnp.maximum(cnt, 1.0)
  mean = agg * inv
  h = lax.dot_general(mean, w1l_ref[...], (((1,), (1,)), ((), ())),
                      preferred_element_type=jnp.float32)
  h += lax.dot_general(x_ref[...], w1r_ref[...], (((1,), (1,)), ((), ())),
                       preferred_element_type=jnp.float32)
  h += b1_ref[...][None, :]
  out_ref[...] = jnp.maximum(h, 0.0)


def _tc_layer2_body(aggp_ref, cntp_ref, h_ref, w2l_ref, b2_ref, w2r_ref,
                    wout_ref, bout_ref, out_ref):
  agg = aggp_ref[0] + aggp_ref[1]
  cnt = cntp_ref[0] + cntp_ref[1]
  inv = 1.0 / jnp.maximum(cnt, 1.0)
  mean = agg * inv
  h = lax.dot_general(mean, w2l_ref[...], (((1,), (1,)), ((), ())),
                      preferred_element_type=jnp.float32)
  h += lax.dot_general(h_ref[...], w2r_ref[...], (((1,), (1,)), ((), ())),
                       preferred_element_type=jnp.float32)
  h += b2_ref[...][None, :]
  h = jnp.maximum(h, 0.0)
  out = lax.dot_general(h, wout_ref[...], (((1,), (1,)), ((), ())),
                        preferred_element_type=jnp.float32)
  out += bout_ref[...][None, :]
  out_ref[...] = out


_agg_spec = pl.BlockSpec((NC, BLK, D), lambda i: (0, i, 0))
_cnt_spec = pl.BlockSpec((NC, BLK, 1), lambda i: (0, i, 0))
_h_spec = pl.BlockSpec((BLK, D), lambda i: (i, 0))
_w_spec = pl.BlockSpec((D, D), lambda i: (0, 0))
_b_spec = pl.BlockSpec((D,), lambda i: (0,))

_tc_layer1 = pl.pallas_call(
    _tc_layer1_body,
    grid=(N_PAD // BLK,),
    in_specs=[_agg_spec, _cnt_spec, _h_spec, _w_spec, _b_spec, _w_spec],
    out_specs=_h_spec,
    out_shape=jax.ShapeDtypeStruct((N_PAD, D), jnp.float32),
)

_tc_layer2 = pl.pallas_call(
    _tc_layer2_body,
    grid=(N_PAD // BLK,),
    in_specs=[_agg_spec, _cnt_spec, _h_spec, _w_spec, _b_spec, _w_spec,
              _w_spec, _b_spec],
    out_specs=_h_spec,
    out_shape=jax.ShapeDtypeStruct((N_PAD, D), jnp.float32),
)


@jax.jit
def kernel(x, edge_index, W1l, b1, W1r, W2l, b2, W2r, Wout, bout):
  edges = edge_index.astype(jnp.int32)
  n_pad_edges = EPW_P - EPW
  src_pad = jnp.zeros((NW, n_pad_edges), jnp.int32)
  dpad = N_NODES + (jnp.arange(NW * n_pad_edges, dtype=jnp.int32)
                    % (N_PAD - N_NODES))
  dst_pad = dpad.reshape(NW, n_pad_edges)
  src = jnp.concatenate([edges[0].reshape(NW, EPW), src_pad], axis=1)
  dst = jnp.concatenate([edges[1].reshape(NW, EPW), dst_pad],
                        axis=1).reshape(NW, NWIN, WIN)
  xp = jnp.zeros((N_PAD, D), jnp.float32).at[:N_NODES].set(x)
  zrow = jnp.zeros((N_PAD, D), jnp.float32)
  zcnt = jnp.zeros((N_PAD,), jnp.float32)

  aggp1, cntp = _sc_aggregate_cnt(src, dst, xp, zrow, zcnt)
  cntp3 = cntp.reshape(NC, N_PAD, 1)
  h1 = _tc_layer1(aggp1, cntp3, xp, W1l, b1, W1r)

  aggp2, _ = _sc_aggregate_nocnt(src, dst, h1, zrow, zcnt)
  out = _tc_layer2(aggp2, cntp3, h1, W2l, b2, W2r, Wout, bout)
  return out[:N_NODES]
